# baseline (speedup 1.0000x reference)
import jax
import jax.numpy as jnp
from jax import lax
from jax.experimental import pallas as pl
from jax.experimental.pallas import tpu as pltpu
from jax.experimental.pallas import tpu_sc as plsc

NC = 2
NS = 16
NW = NC * NS
B = 256
C, H, W = 4, 64, 64
P = C * H
PBLK = 16
TPAD = 1024
BLK_WORDS = P // NW * W * B


def _sc_copy_body(n_hbm, nout_hbm, sem):
    wid = lax.axis_index("s") * NC + lax.axis_index("c")
    pltpu.async_copy(n_hbm.at[wid], nout_hbm.at[wid], sem).wait()


def _sc_noise_copy(n2):
    mesh = plsc.VectorSubcoreMesh(
        core_axis_name="c", subcore_axis_name="s",
        num_cores=NC, num_subcores=NS)
    f = pl.kernel(
        _sc_copy_body,
        out_type=jax.ShapeDtypeStruct((NW, BLK_WORDS), jnp.float32),
        mesh=mesh,
        scratch_types=[
            pltpu.SemaphoreType.DMA,
        ],
    )
    return f(n2)


def _tcg_body(t_ref, tbl_ref, a_ref, am_ref):
    iot = lax.broadcasted_iota(jnp.int32, (TPAD, B), 0)
    oh = jnp.where(iot == t_ref[...], 1.0, 0.0).astype(jnp.float32)
    cf = jnp.dot(tbl_ref[...], oh, preferred_element_type=jnp.float32,
                 precision=lax.Precision.HIGHEST)
    a_ref[...] = cf[0:1, :].reshape(1, 1, B)
    am_ref[...] = cf[1:2, :].reshape(1, 1, B)


def _tc_gather(t2, tbl2):
    return pl.pallas_call(
        _tcg_body,
        out_shape=(jax.ShapeDtypeStruct((1, 1, B), jnp.float32),
                   jax.ShapeDtypeStruct((1, 1, B), jnp.float32)),
    )(t2, tbl2)


def _dense_body(a_ref, am_ref, x_ref, n_ref, o_ref):
    o_ref[...] = a_ref[...] * x_ref[...] + am_ref[...] * n_ref[...]


def _tc_dense(a2, am2, x0T, nT):
    blk = pl.BlockSpec((PBLK, W, B), lambda i: (i, 0, 0))
    cblk = pl.BlockSpec((1, 1, B), lambda i: (0, 0, 0))
    return pl.pallas_call(
        _dense_body,
        grid=(P // PBLK,),
        in_specs=[cblk, cblk, blk, blk],
        out_specs=blk,
        out_shape=jax.ShapeDtypeStruct((P, W, B), jnp.float32),
    )(a2, am2, x0T, nT)


@jax.jit
def _run(x0, t32, noise, ac, am):
    x0T = x0.transpose(1, 2, 3, 0).reshape(P, W, B)
    nT = noise.transpose(1, 2, 3, 0).reshape(P, W, B)
    nout2 = _sc_noise_copy(nT.reshape(NW, BLK_WORDS))
    tbl2 = jnp.stack([jnp.pad(ac, (0, TPAD - ac.shape[0])),
                      jnp.pad(am, (0, TPAD - am.shape[0]))])
    a2, am2 = _tc_gather(t32.reshape(1, B), tbl2)
    outT = _tc_dense(a2, am2, x0T, nT)
    out = outT.reshape(C, H, W, B).transpose(3, 0, 1, 2)
    nout = nout2.reshape(C, H, W, B).transpose(3, 0, 1, 2)
    return out, nout


def kernel(x0, t, noise, sqrt_ac, sqrt_1mac):
    return _run(x0, t.astype(jnp.int32), noise, sqrt_ac, sqrt_1mac)


# speedup vs baseline: 11.3344x; 11.3344x over previous
import jax
import jax.numpy as jnp
from jax import lax
from jax.experimental import pallas as pl
from jax.experimental.pallas import tpu as pltpu
from jax.experimental.pallas import tpu_sc as plsc

NC = 2
NS = 16
NW = NC * NS
B = 256
C, H, W = 4, 64, 64
P = C * H
PBLK = 16
TPAD = 1024
BLK_WORDS = P // NW * W * B


def _sc_copy_body(n_hbm, nout_hbm, b0, b1, b2, b3,
                  i0, i1, i2, i3, o0, o1, o2, o3):
    wid = lax.axis_index("s") * NC + lax.axis_index("c")
    base = wid * (P // NW)
    buf = [b0, b1, b2, b3]
    isem = [i0, i1, i2, i3]
    osem = [o0, o1, o2, o3]
    nb = len(buf)
    in_cp = [None] * nb
    out_cp = [None] * nb

    for s in range(P // NW):
        p = s % nb
        if out_cp[p] is not None:
            out_cp[p].wait()
        in_cp[p] = pltpu.async_copy(n_hbm.at[base + s], buf[p], isem[p])
        in_cp[p].wait()
        out_cp[p] = pltpu.async_copy(buf[p], nout_hbm.at[base + s], osem[p])
    for cp in out_cp:
        cp.wait()


def _sc_noise_copy(nT):
    mesh = plsc.VectorSubcoreMesh(
        core_axis_name="c", subcore_axis_name="s",
        num_cores=NC, num_subcores=NS)
    f = pl.kernel(
        _sc_copy_body,
        out_type=jax.ShapeDtypeStruct((P, W, B), jnp.float32),
        mesh=mesh,
        scratch_types=(
            [pltpu.VMEM((W, B), jnp.float32)] * 4
            + [pltpu.SemaphoreType.DMA] * 8
        ),
    )
    return f(nT)


def _tcg_body(t_ref, tbl_ref, a_ref, am_ref):
    iot = lax.broadcasted_iota(jnp.int32, (TPAD, B), 0)
    oh = jnp.where(iot == t_ref[...], 1.0, 0.0).astype(jnp.float32)
    cf = jnp.dot(tbl_ref[...], oh, preferred_element_type=jnp.float32,
                 precision=lax.Precision.HIGHEST)
    a_ref[...] = cf[0:1, :].reshape(1, 1, B)
    am_ref[...] = cf[1:2, :].reshape(1, 1, B)


def _tc_gather(t2, tbl2):
    return pl.pallas_call(
        _tcg_body,
        out_shape=(jax.ShapeDtypeStruct((1, 1, B), jnp.float32),
                   jax.ShapeDtypeStruct((1, 1, B), jnp.float32)),
    )(t2, tbl2)


def _dense_body(a_ref, am_ref, x_ref, n_ref, o_ref):
    o_ref[...] = a_ref[...] * x_ref[...] + am_ref[...] * n_ref[...]


def _tc_dense(a2, am2, x0T, nT):
    blk = pl.BlockSpec((PBLK, W, B), lambda i: (i, 0, 0))
    cblk = pl.BlockSpec((1, 1, B), lambda i: (0, 0, 0))
    return pl.pallas_call(
        _dense_body,
        grid=(P // PBLK,),
        in_specs=[cblk, cblk, blk, blk],
        out_specs=blk,
        out_shape=jax.ShapeDtypeStruct((P, W, B), jnp.float32),
    )(a2, am2, x0T, nT)


@jax.jit
def _run(x0, t32, noise, ac, am):
    x0T = x0.transpose(1, 2, 3, 0).reshape(P, W, B)
    nT = noise.transpose(1, 2, 3, 0).reshape(P, W, B)
    noutT = _sc_noise_copy(nT)
    tbl2 = jnp.stack([jnp.pad(ac, (0, TPAD - ac.shape[0])),
                      jnp.pad(am, (0, TPAD - am.shape[0]))])
    a2, am2 = _tc_gather(t32.reshape(1, B), tbl2)
    outT = _tc_dense(a2, am2, x0T, nT)
    out = outT.reshape(C, H, W, B).transpose(3, 0, 1, 2)
    nout = noutT.reshape(C, H, W, B).transpose(3, 0, 1, 2)
    return out, nout


def kernel(x0, t, noise, sqrt_ac, sqrt_1mac):
    return _run(x0, t.astype(jnp.int32), noise, sqrt_ac, sqrt_1mac)


# speedup vs baseline: 12.8372x; 1.1326x over previous
import jax
import jax.numpy as jnp
from jax import lax
from jax.experimental import pallas as pl
from jax.experimental.pallas import tpu as pltpu
from jax.experimental.pallas import tpu_sc as plsc

NC = 2
NS = 16
B = 256
C, H, W = 4, 64, 64
P = C * H
PBLK = 32


def _gather_body(t_hbm, ac_hbm, am_hbm, a_out, am_out, t_v, a_v, am_v, csem):
    wid = lax.axis_index("s")

    @pl.when(wid == 0)
    def _():
        pltpu.sync_copy(t_hbm, t_v)
        ccps = []
        for h in range(2):
            sl = pl.ds(h * 128, 128)
            ccps.append(pltpu.async_copy(
                ac_hbm.at[t_v.at[sl]], a_v.at[sl], csem))
            ccps.append(pltpu.async_copy(
                am_hbm.at[t_v.at[sl]], am_v.at[sl], csem))
        for cp in ccps:
            cp.wait()
        cpo = pltpu.async_copy(a_v, a_out, csem)
        cpm = pltpu.async_copy(am_v, am_out, csem)
        cpo.wait()
        cpm.wait()


def _sc_gather(t32, ac, am):
    mesh = plsc.VectorSubcoreMesh(
        core_axis_name="c", subcore_axis_name="s",
        num_cores=1, num_subcores=NS)
    f = pl.kernel(
        _gather_body,
        out_type=(jax.ShapeDtypeStruct((B,), jnp.float32),
                  jax.ShapeDtypeStruct((B,), jnp.float32)),
        mesh=mesh,
        scratch_types=[
            pltpu.VMEM((B,), jnp.int32),
            pltpu.VMEM((B,), jnp.float32),
            pltpu.VMEM((B,), jnp.float32),
            pltpu.SemaphoreType.DMA,
        ],
    )
    return f(t32, ac, am)


def _dense_body(a_ref, am_ref, x_ref, n_ref, o_ref, no_ref):
    n = n_ref[...]
    o_ref[...] = a_ref[...] * x_ref[...] + am_ref[...] * n
    no_ref[...] = n


def _tc_dense(a2, am2, x0T, nT):
    blk = pl.BlockSpec((PBLK, W, B), lambda i: (i, 0, 0))
    cblk = pl.BlockSpec((1, 1, B), lambda i: (0, 0, 0))
    return pl.pallas_call(
        _dense_body,
        grid=(P // PBLK,),
        in_specs=[cblk, cblk, blk, blk],
        out_specs=(blk, blk),
        out_shape=(jax.ShapeDtypeStruct((P, W, B), jnp.float32),
                   jax.ShapeDtypeStruct((P, W, B), jnp.float32)),
    )(a2, am2, x0T, nT)


@jax.jit
def _run(x0, t32, noise, ac, am):
    x0T = x0.transpose(1, 2, 3, 0).reshape(P, W, B)
    nT = noise.transpose(1, 2, 3, 0).reshape(P, W, B)
    a_all, am_all = _sc_gather(t32, ac, am)
    outT, noutT = _tc_dense(a_all.reshape(1, 1, B), am_all.reshape(1, 1, B),
                            x0T, nT)
    out = outT.reshape(C, H, W, B).transpose(3, 0, 1, 2)
    nout = noutT.reshape(C, H, W, B).transpose(3, 0, 1, 2)
    return out, nout


def kernel(x0, t, noise, sqrt_ac, sqrt_1mac):
    return _run(x0, t.astype(jnp.int32), noise, sqrt_ac, sqrt_1mac)


# speedup vs baseline: 13.1122x; 1.0214x over previous
import jax
import jax.numpy as jnp
from jax import lax
from jax.experimental import pallas as pl
from jax.experimental.pallas import tpu as pltpu
from jax.experimental.pallas import tpu_sc as plsc

NC = 2
NS = 16
B = 256
C, H, W = 4, 64, 64
P = C * H
PBLK = 64


def _gather_body(t_hbm, ac_hbm, am_hbm, a_out, am_out, t_v, a_v, am_v, csem):
    wid = lax.axis_index("s")

    @pl.when(wid == 0)
    def _():
        pltpu.sync_copy(t_hbm, t_v)
        ccps = []
        for h in range(2):
            sl = pl.ds(h * 128, 128)
            ccps.append(pltpu.async_copy(
                ac_hbm.at[t_v.at[sl]], a_v.at[sl], csem))
            ccps.append(pltpu.async_copy(
                am_hbm.at[t_v.at[sl]], am_v.at[sl], csem))
        for cp in ccps:
            cp.wait()
        cpo = pltpu.async_copy(a_v, a_out, csem)
        cpm = pltpu.async_copy(am_v, am_out, csem)
        cpo.wait()
        cpm.wait()


def _sc_gather(t32, ac, am):
    mesh = plsc.VectorSubcoreMesh(
        core_axis_name="c", subcore_axis_name="s",
        num_cores=1, num_subcores=NS)
    f = pl.kernel(
        _gather_body,
        out_type=(jax.ShapeDtypeStruct((B,), jnp.float32),
                  jax.ShapeDtypeStruct((B,), jnp.float32)),
        mesh=mesh,
        scratch_types=[
            pltpu.VMEM((B,), jnp.int32),
            pltpu.VMEM((B,), jnp.float32),
            pltpu.VMEM((B,), jnp.float32),
            pltpu.SemaphoreType.DMA,
        ],
    )
    return f(t32, ac, am)


def _dense_body(a_ref, am_ref, x_ref, n_ref, o_ref, no_ref):
    n = n_ref[...]
    o_ref[...] = a_ref[...] * x_ref[...] + am_ref[...] * n
    no_ref[...] = n


def _tc_dense(a2, am2, x0T, nT):
    blk = pl.BlockSpec((PBLK, W, B), lambda i: (i, 0, 0))
    cblk = pl.BlockSpec((1, 1, B), lambda i: (0, 0, 0))
    return pl.pallas_call(
        _dense_body,
        grid=(P // PBLK,),
        in_specs=[cblk, cblk, blk, blk],
        out_specs=(blk, blk),
        out_shape=(jax.ShapeDtypeStruct((P, W, B), jnp.float32),
                   jax.ShapeDtypeStruct((P, W, B), jnp.float32)),
    )(a2, am2, x0T, nT)


@jax.jit
def _run(x0, t32, noise, ac, am):
    x0T = x0.transpose(1, 2, 3, 0).reshape(P, W, B)
    nT = noise.transpose(1, 2, 3, 0).reshape(P, W, B)
    a_all, am_all = _sc_gather(t32, ac, am)
    outT, noutT = _tc_dense(a_all.reshape(1, 1, B), am_all.reshape(1, 1, B),
                            x0T, nT)
    out = outT.reshape(C, H, W, B).transpose(3, 0, 1, 2)
    nout = noutT.reshape(C, H, W, B).transpose(3, 0, 1, 2)
    return out, nout


def kernel(x0, t, noise, sqrt_ac, sqrt_1mac):
    return _run(x0, t.astype(jnp.int32), noise, sqrt_ac, sqrt_1mac)


# speedup vs baseline: 13.1385x; 1.0020x over previous
import jax
import jax.numpy as jnp
from jax import lax
from jax.experimental import pallas as pl
from jax.experimental.pallas import tpu as pltpu
from jax.experimental.pallas import tpu_sc as plsc

NC = 2
NS = 16
B = 256
C, H, W = 4, 64, 64
P = C * H
PBLK = 64


def _gather_body(t_hbm, ac_hbm, am_hbm, a_out, am_out, t_v, a_v, am_v, csem):
    wid = lax.axis_index("s")

    @pl.when(wid == 0)
    def _():
        pltpu.sync_copy(t_hbm, t_v)
        ccps = []
        for h in range(2):
            sl = pl.ds(h * 128, 128)
            ccps.append(pltpu.async_copy(
                ac_hbm.at[t_v.at[sl]], a_v.at[sl], csem))
            ccps.append(pltpu.async_copy(
                am_hbm.at[t_v.at[sl]], am_v.at[sl], csem))
        for cp in ccps:
            cp.wait()
        cpo = pltpu.async_copy(a_v, a_out, csem)
        cpm = pltpu.async_copy(am_v, am_out, csem)
        cpo.wait()
        cpm.wait()


def _sc_gather(t32, ac, am):
    mesh = plsc.VectorSubcoreMesh(
        core_axis_name="c", subcore_axis_name="s",
        num_cores=1, num_subcores=NS)
    f = pl.kernel(
        _gather_body,
        out_type=(jax.ShapeDtypeStruct((B,), jnp.float32),
                  jax.ShapeDtypeStruct((B,), jnp.float32)),
        mesh=mesh,
        scratch_types=[
            pltpu.VMEM((B,), jnp.int32),
            pltpu.VMEM((B,), jnp.float32),
            pltpu.VMEM((B,), jnp.float32),
            pltpu.SemaphoreType.DMA,
        ],
    )
    return f(t32, ac, am)


def _dense_body(a_ref, am_ref, x_ref, n_ref, o_ref, no_ref):
    n = n_ref[...]
    o_ref[...] = a_ref[...] * x_ref[...] + am_ref[...] * n
    no_ref[...] = n


def _tc_dense(a2, am2, x0T, nT):
    blk = pl.BlockSpec((PBLK, W, B), lambda i: (i, 0, 0))
    cblk = pl.BlockSpec((1, 1, B), lambda i: (0, 0, 0))
    return pl.pallas_call(
        _dense_body,
        grid=(P // PBLK,),
        compiler_params=pltpu.CompilerParams(skip_device_barrier=True),
        in_specs=[cblk, cblk, blk, blk],
        out_specs=(blk, blk),
        out_shape=(jax.ShapeDtypeStruct((P, W, B), jnp.float32),
                   jax.ShapeDtypeStruct((P, W, B), jnp.float32)),
    )(a2, am2, x0T, nT)


@jax.jit
def _run(x0, t32, noise, ac, am):
    x0T = x0.transpose(1, 2, 3, 0).reshape(P, W, B)
    nT = noise.transpose(1, 2, 3, 0).reshape(P, W, B)
    a_all, am_all = _sc_gather(t32, ac, am)
    outT, noutT = _tc_dense(a_all.reshape(1, 1, B), am_all.reshape(1, 1, B),
                            x0T, nT)
    out = outT.reshape(C, H, W, B).transpose(3, 0, 1, 2)
    nout = noutT.reshape(C, H, W, B).transpose(3, 0, 1, 2)
    return out, nout


def kernel(x0, t, noise, sqrt_ac, sqrt_1mac):
    return _run(x0, t.astype(jnp.int32), noise, sqrt_ac, sqrt_1mac)
